# baseline (speedup 1.0000x reference)
import jax
import jax.numpy as jnp
import numpy as np
from jax.experimental import pallas as pl

N_NODES = 50000
F_DIM = 80
N_GRAPHS = 128
N_LAYERS = 4

_DEG = np.array([0, 50, 150, 300, 600, 1000, 1500, 2200, 3000, 3800, 4400, 4800, 5000, 4800, 4400, 3800, 3000, 2200, 1500, 1000, 600, 300, 150, 80, 40, 20, 10, 5, 2, 1, 0, 0, 0], dtype=np.float64)
_AVG_LOG = float((np.log(np.arange(_DEG.size) + 1.0) * _DEG).sum() / _DEG.sum())


def _ident(h):
    def k(h_ref, o_ref):
        o_ref[...] = h_ref[...]
    return pl.pallas_call(k, out_shape=jax.ShapeDtypeStruct(h.shape, h.dtype))(h)


def kernel(x, edge_index, batch, emb, Wpre, bpre, Wpost, bpost, Wlin, blin, gamma, beta, Wmlp, bmlp):
    src = edge_index[0]
    dst = edge_index[1]
    h = jnp.zeros((x.shape[0], F_DIM), jnp.float32)
    for f in range(x.shape[1]):
        h = h + emb[f][x[:, f]]
    h = _ident(h)
    ones_e = jnp.ones((src.shape[0],), jnp.float32)
    count = jax.ops.segment_sum(ones_e, dst, num_segments=N_NODES)
    deg = jnp.clip(count, 1.0)
    amp = (jnp.log(deg + 1.0) / _AVG_LOG)[:, None]
    att = (_AVG_LOG / jnp.log(deg + 1.0))[:, None]
    has = (count > 0)[:, None]
    for l in range(N_LAYERS):
        x_h = h
        m = jnp.concatenate([h[dst], h[src]], axis=-1) @ Wpre[l] + bpre[l]
        s = jax.ops.segment_sum(m, dst, num_segments=N_NODES)
        mean = s / deg[:, None]
        s2 = jax.ops.segment_sum(m * m, dst, num_segments=N_NODES)
        var = jax.nn.relu(s2 / deg[:, None] - mean * mean)
        std = jnp.sqrt(var + 1e-5)
        mn = jnp.where(has, jax.ops.segment_min(m, dst, num_segments=N_NODES), 0.0)
        mx = jnp.where(has, jax.ops.segment_max(m, dst, num_segments=N_NODES), 0.0)
        agg = jnp.concatenate([mean, mn, mx, std], axis=-1)
        scaled = jnp.concatenate([agg, agg * amp, agg * att], axis=-1)
        out = jnp.concatenate([h, scaled], axis=-1) @ Wpost[l] + bpost[l]
        out = out @ Wlin[l] + blin[l]
        out = (out / jnp.sqrt(1.0 + 1e-5)) * gamma[l] + beta[l]
        out = jax.nn.relu(out)
        h = x_h + out
    gs = jax.ops.segment_sum(h, batch, num_segments=N_GRAPHS)
    gc = jnp.clip(jax.ops.segment_sum(jnp.ones((h.shape[0],), jnp.float32), batch, num_segments=N_GRAPHS), 1.0)
    pooled = gs / gc[:, None]
    return pooled @ Wmlp + bmlp


# speedup vs baseline: 10.2536x; 10.2536x over previous
import functools

import jax
import jax.numpy as jnp
import numpy as np
from jax import lax
from jax.experimental import pallas as pl
from jax.experimental.pallas import tpu as pltpu
from jax.experimental.pallas import tpu_sc as plsc

N = 50000
E = 800000
F = 80
G = 128
NLAYERS = 4

_DEG = np.array([0, 50, 150, 300, 600, 1000, 1500, 2200, 3000, 3800, 4400, 4800, 5000, 4800, 4400, 3800, 3000, 2200, 1500, 1000, 600, 300, 150, 80, 40, 20, 10, 5, 2, 1, 0, 0, 0], dtype=np.float64)
_AVG = float((np.log(np.arange(_DEG.size) + 1.0) * _DEG).sum() / _DEG.sum())

NW = 32
WIN = 64
K = 128
NBUF = 4
NP = 50048
EPAD = E + NBUF * K + 8

CH = 3128
TCG = NP // CH
PCH = 2000


def _seg_stats(b_tab, srcs, ptr, bounds):
    mesh = plsc.VectorSubcoreMesh(core_axis_name="c", subcore_axis_name="s",
                                  num_cores=2, num_subcores=16)
    outs = tuple(jax.ShapeDtypeStruct((NP, F), jnp.float32) for _ in range(4))

    @functools.partial(
        pl.kernel, out_type=outs, mesh=mesh,
        compiler_params=pltpu.CompilerParams(use_tc_tiling_on_sc=False),
        scratch_types=[
            pltpu.VMEM((NBUF, K), jnp.int32),
            pltpu.VMEM((NBUF * K, F), jnp.float32),
            pltpu.VMEM((WIN, F), jnp.float32),
            pltpu.VMEM((WIN, F), jnp.float32),
            pltpu.VMEM((WIN, F), jnp.float32),
            pltpu.VMEM((WIN, F), jnp.float32),
            pltpu.VMEM((WIN + 16,), jnp.int32),
            pltpu.VMEM((48,), jnp.int32),
            pltpu.SemaphoreType.DMA((NBUF,)),
        ])
    def agg(b_hbm, src_hbm, ptr_hbm, bnd_hbm, ssum_hbm, ssq_hbm, smn_hbm, smx_hbm,
            idx_v, rows_v, ssum_v, ssq_v, smn_v, smx_v, ptr_s, bnd_s, sems):
        wid = lax.axis_index("s") * 2 + lax.axis_index("c")
        pltpu.sync_copy(bnd_hbm, bnd_s)
        bv = bnd_s[pl.ds(wid, 16)]
        n0 = bv[0]
        n1 = bv[1]

        @pl.loop(n0, n1, step=WIN)
        def _window(w0):
            w0 = pl.multiple_of(w0, WIN)
            pltpu.sync_copy(ptr_hbm.at[pl.ds(w0, WIN + 16)], ptr_s)
            e_begin = ptr_s[pl.ds(0, 16)][0]
            e_end = ptr_s[pl.ds(WIN, 16)][0]
            ea = (e_begin // 8) * 8
            nc = (e_end - ea + (K - 1)) // K

            zv = jnp.zeros((16,), jnp.float32)

            @pl.loop(0, WIN)
            def _zero(n):
                for j in range(5):
                    sl = pl.ds(16 * j, 16)
                    ssum_v[n, sl] = zv
                    ssq_v[n, sl] = zv
                    smn_v[n, sl] = zv
                    smx_v[n, sl] = zv

            def issue(c, b):
                e0 = pl.multiple_of(ea + c * K, 8)
                pltpu.sync_copy(src_hbm.at[pl.ds(e0, K)], idx_v.at[b])
                pltpu.async_copy(b_hbm.at[idx_v.at[b]],
                                 rows_v.at[pl.ds(b * K, K)], sems.at[b])

            def wait(b):
                pltpu.make_async_copy(b_hbm.at[pl.ds(0, K)],
                                      rows_v.at[pl.ds(b * K, K)],
                                      sems.at[b]).wait()

            def process(c, b):
                chunk_lo = ea + c * K
                chunk_hi = chunk_lo + K

                @pl.loop(0, WIN)
                def _node(ln):
                    pv = ptr_s[pl.ds(ln, 16)]
                    e_lo = jnp.maximum(pv[0], chunk_lo)
                    e_hi = jnp.minimum(pv[1], chunk_hi)

                    @pl.when(e_lo < e_hi)
                    def _():
                        st = []
                        for j in range(5):
                            sl = pl.ds(16 * j, 16)
                            st += [ssum_v[ln, sl], ssq_v[ln, sl],
                                   smn_v[ln, sl], smx_v[ln, sl]]
                        fresh = pv[0] >= chunk_lo
                        big = jnp.full((16,), 3.0e38, jnp.float32)
                        for j in range(5):
                            st[4 * j + 2] = jnp.where(fresh, big, st[4 * j + 2])
                            st[4 * j + 3] = jnp.where(fresh, -big, st[4 * j + 3])

                        @pl.loop(e_lo, e_hi, init_carry=tuple(st))
                        def res(e, acc):
                            acc = list(acc)
                            p = e - chunk_lo + b * K
                            for j in range(5):
                                r = rows_v[p, pl.ds(16 * j, 16)]
                                acc[4 * j] = acc[4 * j] + r
                                acc[4 * j + 1] = acc[4 * j + 1] + r * r
                                acc[4 * j + 2] = jnp.minimum(acc[4 * j + 2], r)
                                acc[4 * j + 3] = jnp.maximum(acc[4 * j + 3], r)
                            return tuple(acc)

                        for j in range(5):
                            sl = pl.ds(16 * j, 16)
                            ssum_v[ln, sl] = res[4 * j]
                            ssq_v[ln, sl] = res[4 * j + 1]
                            smn_v[ln, sl] = res[4 * j + 2]
                            smx_v[ln, sl] = res[4 * j + 3]

            for b in range(NBUF):
                @pl.when(b < nc)
                def _(b=b):
                    issue(jnp.int32(b), b)

            @pl.loop(0, nc, step=NBUF)
            def _grp(g):
                for b in range(NBUF):
                    c = g + b

                    @pl.when(c < nc)
                    def _(c=c, b=b):
                        wait(b)
                        process(c, b)

                        @pl.when(c + NBUF < nc)
                        def _(c=c, b=b):
                            issue(c + NBUF, b)

            pltpu.sync_copy(ssum_v, ssum_hbm.at[pl.ds(w0, WIN)])
            pltpu.sync_copy(ssq_v, ssq_hbm.at[pl.ds(w0, WIN)])
            pltpu.sync_copy(smn_v, smn_hbm.at[pl.ds(w0, WIN)])
            pltpu.sync_copy(smx_v, smx_hbm.at[pl.ds(w0, WIN)])

    return agg(b_tab, srcs, ptr, bounds)


def _enc_pre(xp, Demb, base, Wa, Wb, bpre0):
    def body(x_ref, d_ref, base_ref, wa_ref, wb_ref, bp_ref, h_ref, a_ref, b_ref):
        xf = x_ref[...].astype(jnp.float32)
        h = xf @ d_ref[...] + base_ref[...]
        h_ref[...] = h
        a_ref[...] = h @ wa_ref[...]
        b_ref[...] = h @ wb_ref[...] + bp_ref[...]

    full = lambda s: pl.BlockSpec(s, lambda i: (0, 0))
    return pl.pallas_call(
        body,
        grid=(TCG,),
        in_specs=[
            pl.BlockSpec((CH, 9), lambda i: (i, 0)),
            full((9, F)), full((1, F)), full((F, F)), full((F, F)), full((1, F)),
        ],
        out_specs=[pl.BlockSpec((CH, F), lambda i: (i, 0))] * 3,
        out_shape=[jax.ShapeDtypeStruct((NP, F), jnp.float32)] * 3,
    )(xp, Demb, base, Wa, Wb, bpre0)


def _post(h, a, sb, sb2, smn, smx, cnt, P0, Q1, Q2, Q3, W2, b2, Wa, Wb, bpre, *, make_pre):
    def body(h_ref, a_ref, sb_ref, sb2_ref, smn_ref, smx_ref, cnt_ref,
             p0_ref, q1_ref, q2_ref, q3_ref, w2_ref, b2_ref, wa_ref, wb_ref, bp_ref,
             ho_ref, ao_ref, bo_ref):
        cnt = cnt_ref[...]
        deg = jnp.maximum(cnt, 1.0)
        invdeg = 1.0 / deg
        cda = cnt * invdeg
        lg = jnp.log(deg + 1.0) * (1.0 / _AVG)
        att = 1.0 / lg
        has = (cnt > 0.0).astype(jnp.float32)
        h = h_ref[...]
        av = a_ref[...]
        sbv = sb_ref[...] * invdeg
        mean = cda * av + sbv
        ex2 = cda * (av * av) + 2.0 * av * sbv + sb2_ref[...] * invdeg
        var = jnp.maximum(ex2 - mean * mean, 0.0)
        std = jnp.sqrt(var + 1e-5)
        mn = has * (av + smn_ref[...])
        mx = has * (av + smx_ref[...])
        agg = jnp.concatenate([mean, mn, mx, std], axis=-1)
        z = (h @ p0_ref[...] + agg @ q1_ref[...]
             + lg * (agg @ q2_ref[...]) + att * (agg @ q3_ref[...]))
        out = jnp.maximum(z @ w2_ref[...] + b2_ref[...], 0.0)
        hn = h + out
        ho_ref[...] = hn
        if make_pre:
            ao_ref[...] = hn @ wa_ref[...]
            bo_ref[...] = hn @ wb_ref[...] + bp_ref[...]

    full = lambda s: pl.BlockSpec(s, lambda i: (0, 0))
    nblk = pl.BlockSpec((CH, F), lambda i: (i, 0))
    n_out = 3 if make_pre else 1
    return pl.pallas_call(
        body,
        grid=(TCG,),
        in_specs=[nblk] * 6 + [pl.BlockSpec((CH, 1), lambda i: (i, 0)),
                               full((F, F)), full((4 * F, F)), full((4 * F, F)),
                               full((4 * F, F)), full((F, F)), full((1, F)),
                               full((F, F)), full((F, F)), full((1, F))],
        out_specs=[nblk] * 3,
        out_shape=[jax.ShapeDtypeStruct((NP, F), jnp.float32)] * 3,
    )(h, a, sb, sb2, smn, smx, cnt, P0, Q1, Q2, Q3, W2, b2, Wa, Wb, bpre)[:n_out]


def _pool(h, batch2, Wmlp, bmlp):
    def body(h_ref, b_ref, wm_ref, bm_ref, o_ref, acc_ref, cnt_ref):
        i = pl.program_id(0)

        @pl.when(i == 0)
        def _():
            acc_ref[...] = jnp.zeros_like(acc_ref)
            cnt_ref[...] = jnp.zeros_like(cnt_ref)

        onehot = (b_ref[...] == lax.broadcasted_iota(jnp.int32, (1, G), 1)
                  ).astype(jnp.float32)
        acc_ref[...] += lax.dot_general(onehot, h_ref[...],
                                        (((0,), (0,)), ((), ())),
                                        preferred_element_type=jnp.float32)
        cnt_ref[...] += lax.dot_general(onehot, jnp.ones((PCH, 1), jnp.float32),
                                        (((0,), (0,)), ((), ())),
                                        preferred_element_type=jnp.float32)

        @pl.when(i == (N // PCH) - 1)
        def _():
            pooled = acc_ref[...] / jnp.maximum(cnt_ref[...], 1.0)
            o_ref[...] = pooled @ wm_ref[...] + bm_ref[...]

    return pl.pallas_call(
        body,
        grid=(N // PCH,),
        in_specs=[pl.BlockSpec((PCH, F), lambda i: (i, 0)),
                  pl.BlockSpec((PCH, 1), lambda i: (i, 0)),
                  pl.BlockSpec((F, 1), lambda i: (0, 0)),
                  pl.BlockSpec((1, 1), lambda i: (0, 0))],
        out_specs=pl.BlockSpec((G, 1), lambda i: (0, 0)),
        out_shape=jax.ShapeDtypeStruct((G, 1), jnp.float32),
        scratch_shapes=[pltpu.VMEM((G, F), jnp.float32),
                        pltpu.VMEM((G, 1), jnp.float32)],
    )(h, batch2, Wmlp, bmlp)


def kernel(x, edge_index, batch, emb, Wpre, bpre, Wpost, bpost, Wlin, blin,
           gamma, beta, Wmlp, bmlp):
    src = edge_index[0].astype(jnp.int32)
    dst = edge_index[1].astype(jnp.int32)

    counts = jnp.zeros((NP,), jnp.int32).at[dst].add(1)
    ptr = jnp.concatenate([jnp.zeros((1,), jnp.int32),
                           jnp.cumsum(counts, dtype=jnp.int32),
                           jnp.full((15,), E, jnp.int32)])
    order = jnp.argsort(dst)
    src_sorted = jnp.zeros((EPAD,), jnp.int32).at[:E].set(src[order])
    targets = (jnp.arange(33, dtype=jnp.int32) * (E // NW)).astype(jnp.int32)
    bnd = jnp.searchsorted(ptr[:NP + 1], targets, side="left").astype(jnp.int32)
    bnd = (bnd // WIN) * WIN
    bnd = bnd.at[0].set(0).at[32].set(NP)
    bnd = lax.cummax(bnd)
    bounds = jnp.concatenate([bnd, jnp.zeros((15,), jnp.int32)])

    cnt_f = counts[:, None].astype(jnp.float32)

    base = emb[:, 0, :].sum(0)[None, :]
    Demb = emb[:, 1, :] - emb[:, 0, :]
    sbn = gamma / np.sqrt(1.0 + 1e-5)
    Wa = Wpre[:, :F, :]
    Wb = Wpre[:, F:, :]
    P0 = Wpost[:, :F, :]
    Q1 = Wpost[:, F:5 * F, :]
    Q2 = Wpost[:, 5 * F:9 * F, :]
    Q3 = Wpost[:, 9 * F:, :]
    W2 = Wlin * sbn[:, None, :]
    b2 = (jnp.einsum("lf,lfg->lg", bpost, Wlin) + blin) * sbn + beta

    xp = jnp.zeros((NP, 9), jnp.int32).at[:N].set(x.astype(jnp.int32))

    h, a, b = _enc_pre(xp, Demb, base, Wa[0], Wb[0], bpre[0][None, :])
    for l in range(NLAYERS):
        sb, sb2, smn, smx = _seg_stats(b, src_sorted, ptr, bounds)
        mp = l < NLAYERS - 1
        ln = l + 1 if mp else l
        res = _post(h, a, sb, sb2, smn, smx, cnt_f,
                    P0[l], Q1[l], Q2[l], Q3[l], W2[l], b2[l][None, :],
                    Wa[ln], Wb[ln], bpre[ln][None, :], make_pre=mp)
        if mp:
            h, a, b = res
        else:
            h = res[0]

    return _pool(h[:N], batch[:, None].astype(jnp.int32),
                 Wmlp, bmlp[None, :])
